# baseline (speedup 1.0000x reference)
import functools

import jax
import jax.numpy as jnp
from jax import lax
from jax.experimental import pallas as pl
from jax.experimental.pallas import tpu as pltpu
from jax.experimental.pallas import tpu_sc as plsc

MASKED = -1000000000000.0

_NC = 2
_NS = 16
_NW = _NC * _NS
_LANES = 16


def _sc_mask_builder(v, n, n_workers, chunk):
    mesh = plsc.VectorSubcoreMesh(core_axis_name="c", subcore_axis_name="s")

    @functools.partial(
        pl.kernel,
        mesh=mesh,
        out_type=jax.ShapeDtypeStruct((v,), jnp.float32),
        scratch_types=[
            pltpu.VMEM((n,), jnp.int32),
            pltpu.VMEM((chunk,), jnp.float32),
        ],
        compiler_params=pltpu.CompilerParams(needs_layout_passes=False),
    )
    def sc_mask(ids_hbm, mask_hbm, ids_v, chunk_v):
        c = lax.axis_index("c")
        s = lax.axis_index("s")
        wid = s * _NC + c

        @pl.when(wid < n_workers)
        def _():
            base = pl.multiple_of(wid * chunk, 8)
            zeros16 = jnp.zeros((_LANES,), jnp.float32)
            ones16 = jnp.ones((_LANES,), jnp.float32)

            n_full = n // _LANES
            tail = n % _LANES

            pltpu.sync_copy(ids_hbm, ids_v)

            def zero_body(i, _):
                chunk_v[pl.ds(i * _LANES, _LANES)] = zeros16
                return 0

            lax.fori_loop(0, chunk // _LANES, zero_body, 0)

            def scatter_at(off):
                ids16 = ids_v[pl.ds(off, _LANES)]
                local = ids16 - base
                in_range = (local >= 0) & (local < chunk)
                safe = jnp.where(in_range, local, 0)
                plsc.store_scatter(chunk_v, [safe], ones16, mask=in_range)

            def scatter_body(j, _):
                scatter_at(j * _LANES)
                return 0

            lax.fori_loop(0, n_full, scatter_body, 0)
            if tail:
                scatter_at(n - _LANES)

            pltpu.sync_copy(chunk_v, mask_hbm.at[pl.ds(base, chunk)])

    return sc_mask


def _tc_select_body(mask_ref, log_ref, out_ref):
    b = log_ref.shape[1]
    ones_row = jnp.ones((1, b), jnp.float32)
    dn = (((0,), (0,)), ((), ()))
    mb = jax.lax.dot_general(mask_ref[0], ones_row, dn,
                             preferred_element_type=jnp.float32)
    out_ref[...] = jnp.where(mb != 0.0, log_ref[...], MASKED)


def kernel(verb_logits, noun_logits, seen_verb_ids, seen_noun_ids):
    b, v = verb_logits.shape
    n = seen_verb_ids.shape[0]

    n_workers = 1
    for w in range(_NW, 0, -1):
        if v % w == 0 and (v // w) % 8 == 0:
            n_workers = w
            break
    chunk = v // n_workers

    sc_mask = _sc_mask_builder(v, n, n_workers, chunk)

    rows = 2000
    grid = v // rows

    tc_select = pl.pallas_call(
        _tc_select_body,
        grid=(grid,),
        in_specs=[
            pl.BlockSpec((1, 1, rows), lambda i: (i, 0, 0)),
            pl.BlockSpec((rows, b), lambda i: (i, 0)),
        ],
        out_specs=pl.BlockSpec((rows, b), lambda i: (i, 0)),
        out_shape=jax.ShapeDtypeStruct((v, b), jnp.float32),
        compiler_params=pltpu.CompilerParams(
            dimension_semantics=("arbitrary",),
        ),
    )

    outs = []
    for logits, ids in ((verb_logits, seen_verb_ids), (noun_logits, seen_noun_ids)):
        mask = sc_mask(ids)
        mask3d = mask.reshape(grid, 1, rows)
        outs.append(tc_select(mask3d, logits.T).T)

    return (outs[0], outs[1])


# speedup vs baseline: 1.4897x; 1.4897x over previous
import functools

import jax
import jax.numpy as jnp
from jax import lax
from jax.experimental import pallas as pl
from jax.experimental.pallas import tpu as pltpu
from jax.experimental.pallas import tpu_sc as plsc

MASKED = -1000000000000.0

_NC = 2
_NS = 16
_LANES = 16


def _sc_mask_builder(v, n, chunk, tail_chunk):
    mesh = plsc.VectorSubcoreMesh(core_axis_name="c", subcore_axis_name="s")

    @functools.partial(
        pl.kernel,
        mesh=mesh,
        out_type=(
            jax.ShapeDtypeStruct((v,), jnp.float32),
            jax.ShapeDtypeStruct((v,), jnp.float32),
        ),
        scratch_types=[
            pltpu.VMEM((n,), jnp.int32),
            pltpu.VMEM((chunk,), jnp.float32),
        ],
        compiler_params=pltpu.CompilerParams(needs_layout_passes=False),
    )
    def sc_mask(vids_hbm, nids_hbm, vmask_hbm, nmask_hbm, ids_v, chunk_v):
        c = lax.axis_index("c")
        s = lax.axis_index("s")

        zeros16 = jnp.zeros((_LANES,), jnp.float32)
        ones16 = jnp.ones((_LANES,), jnp.float32)
        n_full = n // _LANES
        tail = n % _LANES

        def build(ids_hbm, mask_hbm):
            base = pl.multiple_of(s * chunk, 8)
            pltpu.sync_copy(ids_hbm, ids_v)

            def make_pass(width):
                def zero_body(i, _):
                    chunk_v[pl.ds(i * _LANES, _LANES)] = zeros16
                    return 0

                lax.fori_loop(0, width // _LANES, zero_body, 0)

                def scatter_at(off):
                    ids16 = ids_v[pl.ds(off, _LANES)]
                    local = ids16 - base
                    in_range = (local >= 0) & (local < width)
                    safe = jnp.where(in_range, local, 0)
                    plsc.store_scatter(chunk_v, [safe], ones16, mask=in_range)

                def scatter_body(j, _):
                    scatter_at(j * _LANES)
                    return 0

                lax.fori_loop(0, n_full, scatter_body, 0)
                if tail:
                    scatter_at(n - _LANES)

                pltpu.sync_copy(
                    chunk_v.at[pl.ds(0, width)], mask_hbm.at[pl.ds(base, width)]
                )

            @pl.when(s < _NS - 1)
            def _():
                make_pass(chunk)

            @pl.when(s == _NS - 1)
            def _():
                make_pass(tail_chunk)

        @pl.when(c == 0)
        def _():
            build(vids_hbm, vmask_hbm)

        @pl.when(c == 1)
        def _():
            build(nids_hbm, nmask_hbm)

    return sc_mask


def _tc_select_body(vmask_ref, nmask_ref, vlog_ref, nlog_ref, vout_ref, nout_ref):
    b = vlog_ref.shape[1]
    ones_row = jnp.ones((1, b), jnp.float32)
    dn = (((0,), (0,)), ((), ()))
    vb = jax.lax.dot_general(vmask_ref[0], ones_row, dn,
                             preferred_element_type=jnp.float32)
    nb = jax.lax.dot_general(nmask_ref[0], ones_row, dn,
                             preferred_element_type=jnp.float32)
    vout_ref[...] = jnp.where(vb != 0.0, vlog_ref[...], MASKED)
    nout_ref[...] = jnp.where(nb != 0.0, nlog_ref[...], MASKED)


def kernel(verb_logits, noun_logits, seen_verb_ids, seen_noun_ids):
    b, v = verb_logits.shape
    n = seen_verb_ids.shape[0]

    chunk = ((v + _NS - 1) // _NS + 7) // 8 * 8
    tail_chunk = v - (_NS - 1) * chunk
    assert tail_chunk > 0 and tail_chunk % 8 == 0

    vmask, nmask = _sc_mask_builder(v, n, chunk, tail_chunk)(
        seen_verb_ids, seen_noun_ids
    )

    vlog_t = verb_logits.T
    nlog_t = noun_logits.T

    rows = 4000
    grid = v // rows

    vmask3d = vmask.reshape(grid, 1, rows)
    nmask3d = nmask.reshape(grid, 1, rows)

    out = pl.pallas_call(
        _tc_select_body,
        grid=(grid,),
        in_specs=[
            pl.BlockSpec((1, 1, rows), lambda i: (i, 0, 0)),
            pl.BlockSpec((1, 1, rows), lambda i: (i, 0, 0)),
            pl.BlockSpec((rows, b), lambda i: (i, 0)),
            pl.BlockSpec((rows, b), lambda i: (i, 0)),
        ],
        out_specs=[
            pl.BlockSpec((rows, b), lambda i: (i, 0)),
            pl.BlockSpec((rows, b), lambda i: (i, 0)),
        ],
        out_shape=[
            jax.ShapeDtypeStruct((v, b), jnp.float32),
            jax.ShapeDtypeStruct((v, b), jnp.float32),
        ],
        compiler_params=pltpu.CompilerParams(
            dimension_semantics=("parallel",),
        ),
    )(vmask3d, nmask3d, vlog_t, nlog_t)

    return (out[0].T, out[1].T)


# speedup vs baseline: 1.6067x; 1.0786x over previous
import functools

import jax
import jax.numpy as jnp
from jax import lax
from jax.experimental import pallas as pl
from jax.experimental.pallas import tpu as pltpu
from jax.experimental.pallas import tpu_sc as plsc

MASKED = -1000000000000.0

_NC = 2
_NS = 16
_LANES = 16


def _sc_mask_builder(v_pad, n, chunk):
    mesh = plsc.VectorSubcoreMesh(core_axis_name="c", subcore_axis_name="s")

    @functools.partial(
        pl.kernel,
        mesh=mesh,
        out_type=(
            jax.ShapeDtypeStruct((v_pad,), jnp.float32),
            jax.ShapeDtypeStruct((v_pad,), jnp.float32),
        ),
        scratch_types=[
            pltpu.VMEM((n,), jnp.int32),
            pltpu.VMEM((chunk,), jnp.float32),
            pltpu.SemaphoreType.DMA,
        ],
        compiler_params=pltpu.CompilerParams(needs_layout_passes=False),
    )
    def sc_mask(vids_hbm, nids_hbm, vmask_hbm, nmask_hbm, ids_v, chunk_v, sem):
        c = lax.axis_index("c")
        s = lax.axis_index("s")

        zeros16 = jnp.zeros((_LANES,), jnp.float32)
        ones16 = jnp.ones((_LANES,), jnp.float32)
        n_full = n // _LANES
        tail = n % _LANES
        base = pl.multiple_of(s * chunk, 8)

        def build(ids_hbm, mask_hbm):
            ids_cp = pltpu.async_copy(ids_hbm, ids_v, sem)

            def zero_body(i, _):
                chunk_v[pl.ds(i * _LANES, _LANES)] = zeros16
                return 0

            lax.fori_loop(0, chunk // _LANES, zero_body, 0)
            ids_cp.wait()

            def scatter_at(off):
                ids16 = ids_v[pl.ds(off, _LANES)]
                local = ids16 - base
                in_range = (local >= 0) & (local < chunk)
                safe = jnp.where(in_range, local, 0)
                plsc.store_scatter(chunk_v, [safe], ones16, mask=in_range)

            def scatter_body(j, _):
                scatter_at(j * _LANES)
                return 0

            lax.fori_loop(0, n_full, scatter_body, 0)
            if tail:
                scatter_at(n - _LANES)

            pltpu.sync_copy(chunk_v, mask_hbm.at[pl.ds(base, chunk)])

        @pl.when(c == 0)
        def _():
            build(vids_hbm, vmask_hbm)

        @pl.when(c == 1)
        def _():
            build(nids_hbm, nmask_hbm)

    return sc_mask


def _tc_select_body(vmask_ref, nmask_ref, vlog_ref, nlog_ref, vout_ref, nout_ref):
    b = vlog_ref.shape[1]
    rows = vmask_ref.shape[0]
    ones_row = jnp.ones((1, b), jnp.float32)
    dn = (((0,), (0,)), ((), ()))
    vb = jax.lax.dot_general(vmask_ref[...].reshape(1, rows), ones_row, dn,
                             preferred_element_type=jnp.float32)
    nb = jax.lax.dot_general(nmask_ref[...].reshape(1, rows), ones_row, dn,
                             preferred_element_type=jnp.float32)
    vout_ref[...] = jnp.where(vb != 0.0, vlog_ref[...], MASKED)
    nout_ref[...] = jnp.where(nb != 0.0, nlog_ref[...], MASKED)


def kernel(verb_logits, noun_logits, seen_verb_ids, seen_noun_ids):
    b, v = verb_logits.shape
    n = seen_verb_ids.shape[0]

    rows = 4096
    grid = (v + rows - 1) // rows
    v_pad = grid * rows
    chunk = v_pad // _NS
    assert chunk % 8 == 0

    vmask, nmask = _sc_mask_builder(v_pad, n, chunk)(seen_verb_ids, seen_noun_ids)

    vlog_t = verb_logits.T
    nlog_t = noun_logits.T

    out = pl.pallas_call(
        _tc_select_body,
        grid=(grid,),
        in_specs=[
            pl.BlockSpec((rows,), lambda i: (i,)),
            pl.BlockSpec((rows,), lambda i: (i,)),
            pl.BlockSpec((rows, b), lambda i: (i, 0)),
            pl.BlockSpec((rows, b), lambda i: (i, 0)),
        ],
        out_specs=[
            pl.BlockSpec((rows, b), lambda i: (i, 0)),
            pl.BlockSpec((rows, b), lambda i: (i, 0)),
        ],
        out_shape=[
            jax.ShapeDtypeStruct((v, b), jnp.float32),
            jax.ShapeDtypeStruct((v, b), jnp.float32),
        ],
        compiler_params=pltpu.CompilerParams(
            dimension_semantics=("parallel",),
        ),
    )(vmask, nmask, vlog_t, nlog_t)

    return (out[0].T, out[1].T)


# speedup vs baseline: 1.6468x; 1.0249x over previous
import functools

import jax
import jax.numpy as jnp
from jax import lax
from jax.experimental import pallas as pl
from jax.experimental.pallas import tpu as pltpu
from jax.experimental.pallas import tpu_sc as plsc

MASKED = -1000000000000.0

_NC = 2
_NS = 16
_LANES = 16


def _sc_mask_builder(v_pad, n, chunk):
    mesh = plsc.VectorSubcoreMesh(core_axis_name="c", subcore_axis_name="s")

    @functools.partial(
        pl.kernel,
        mesh=mesh,
        out_type=(
            jax.ShapeDtypeStruct((v_pad,), jnp.float32),
            jax.ShapeDtypeStruct((v_pad,), jnp.float32),
        ),
        scratch_types=[
            pltpu.VMEM((n,), jnp.int32),
            pltpu.VMEM((chunk,), jnp.float32),
            pltpu.SemaphoreType.DMA,
        ],
        compiler_params=pltpu.CompilerParams(needs_layout_passes=False),
    )
    def sc_mask(vids_hbm, nids_hbm, vmask_hbm, nmask_hbm, ids_v, chunk_v, sem):
        c = lax.axis_index("c")
        s = lax.axis_index("s")

        zeros16 = jnp.zeros((_LANES,), jnp.float32)
        ones16 = jnp.ones((_LANES,), jnp.float32)
        n_full = n // _LANES
        tail = n % _LANES
        base = pl.multiple_of(s * chunk, 8)

        def build(ids_hbm, mask_hbm):
            ids_cp = pltpu.async_copy(ids_hbm, ids_v, sem)

            def zero_body(i, _):
                chunk_v[pl.ds(i * _LANES, _LANES)] = zeros16
                return 0

            lax.fori_loop(0, chunk // _LANES, zero_body, 0)
            ids_cp.wait()

            def scatter_at(off):
                ids16 = ids_v[pl.ds(off, _LANES)]
                local = ids16 - base
                in_range = (local >= 0) & (local < chunk)
                safe = jnp.where(in_range, local, 0)
                plsc.store_scatter(chunk_v, [safe], ones16, mask=in_range)

            def scatter_body(j, _):
                scatter_at(j * _LANES)
                return 0

            lax.fori_loop(0, n_full, scatter_body, 0)
            if tail:
                scatter_at(n - _LANES)

            pltpu.sync_copy(chunk_v, mask_hbm.at[pl.ds(base, chunk)])

        @pl.when(c == 0)
        def _():
            build(vids_hbm, vmask_hbm)

        @pl.when(c == 1)
        def _():
            build(nids_hbm, nmask_hbm)

    return sc_mask


def _tc_select_body(vmask_ref, nmask_ref, vlog_ref, nlog_ref, vout_ref, nout_ref):
    b = vlog_ref.shape[1]
    rows = vmask_ref.shape[0]
    ones_row = jnp.ones((1, b), jnp.float32)
    dn = (((0,), (0,)), ((), ()))
    vb = jax.lax.dot_general(vmask_ref[...].reshape(1, rows), ones_row, dn,
                             preferred_element_type=jnp.float32)
    nb = jax.lax.dot_general(nmask_ref[...].reshape(1, rows), ones_row, dn,
                             preferred_element_type=jnp.float32)
    vout_ref[...] = jnp.where(vb != 0.0, vlog_ref[...], MASKED)
    nout_ref[...] = jnp.where(nb != 0.0, nlog_ref[...], MASKED)


def kernel(verb_logits, noun_logits, seen_verb_ids, seen_noun_ids):
    b, v = verb_logits.shape
    n = seen_verb_ids.shape[0]

    rows = 8192
    grid = (v + rows - 1) // rows
    v_pad = grid * rows
    chunk = v_pad // _NS
    assert chunk % 8 == 0

    vmask, nmask = _sc_mask_builder(v_pad, n, chunk)(seen_verb_ids, seen_noun_ids)

    vlog_t = verb_logits.T
    nlog_t = noun_logits.T

    out = pl.pallas_call(
        _tc_select_body,
        grid=(grid,),
        in_specs=[
            pl.BlockSpec((rows,), lambda i: (i,)),
            pl.BlockSpec((rows,), lambda i: (i,)),
            pl.BlockSpec((rows, b), lambda i: (i, 0)),
            pl.BlockSpec((rows, b), lambda i: (i, 0)),
        ],
        out_specs=[
            pl.BlockSpec((rows, b), lambda i: (i, 0)),
            pl.BlockSpec((rows, b), lambda i: (i, 0)),
        ],
        out_shape=[
            jax.ShapeDtypeStruct((v, b), jnp.float32),
            jax.ShapeDtypeStruct((v, b), jnp.float32),
        ],
        compiler_params=pltpu.CompilerParams(
            dimension_semantics=("parallel",),
        ),
    )(vmask, nmask, vlog_t, nlog_t)

    return (out[0].T, out[1].T)
